# baseline (speedup 1.0000x reference)
import jax
import jax.numpy as jnp
from jax.experimental import pallas as pl
from jax.experimental.pallas import tpu as pltpu
from jax.experimental.pallas import tpu_sc as plsc

EPS = 1e-12
GW = 128
BK = 8


def _sc_gather(table, idx_flat):
    n = idx_flat.shape[1]
    dim = table.shape[1]

    @pl.kernel(
        out_type=jax.ShapeDtypeStruct((n, dim), table.dtype),
        mesh=plsc.VectorSubcoreMesh(core_axis_name="core",
                                    subcore_axis_name="subcore"),
    )
    def k(tab_hbm, i_hbm, o_hbm):
        def body(i_vmem, o_vmem):
            pltpu.sync_copy(tab_hbm.at[i_vmem.at[0]], o_vmem)

        pltpu.emit_pipeline(
            body,
            grid=(n // GW,),
            in_specs=[pl.BlockSpec((1, GW), index_map=lambda i: (0, i))],
            out_specs=[pl.BlockSpec((GW, dim), index_map=lambda i: (i, 0))],
            core_axis_name=("core", "subcore"),
            dimension_semantics=(pltpu.PARALLEL,),
        )(i_hbm, o_hbm)

    return k(table, idx_flat)


def _tc_layernorm(x, pos, gamma, beta):
    b, sig, dim = x.shape

    def body(x_ref, pos_ref, g_ref, bt_ref, o_ref):
        v = x_ref[...] + pos_ref[...]
        mean = jnp.mean(v, axis=-1, keepdims=True)
        c = v - mean
        var = jnp.mean(c * c, axis=-1, keepdims=True)
        o_ref[...] = c * jax.lax.rsqrt(var + EPS) * g_ref[...] + bt_ref[...]

    return pl.pallas_call(
        body,
        grid=(b // BK,),
        in_specs=[
            pl.BlockSpec((BK, sig, dim), lambda i: (i, 0, 0)),
            pl.BlockSpec((1, sig, dim), lambda i: (0, 0, 0)),
            pl.BlockSpec((1, 1, dim), lambda i: (0, 0, 0)),
            pl.BlockSpec((1, 1, dim), lambda i: (0, 0, 0)),
        ],
        out_specs=pl.BlockSpec((BK, sig, dim), lambda i: (i, 0, 0)),
        out_shape=jax.ShapeDtypeStruct((b, sig, dim), jnp.float32),
    )(x, pos, gamma, beta)


def kernel(news_batch, word_embeddings, pos_embedding, gamma, beta):
    b, sig = news_batch.shape
    vocab, dim = word_embeddings.shape
    half = dim // 2
    table2 = word_embeddings.reshape(2 * vocab, half)
    idx = news_batch.reshape(b * sig).astype(jnp.int32)
    idx2 = jnp.stack([2 * idx, 2 * idx + 1], axis=-1).reshape(1, 2 * b * sig)
    gathered = _sc_gather(table2, idx2)
    x = gathered.reshape(b, sig, dim)
    return _tc_layernorm(x, pos_embedding,
                         gamma.reshape(1, 1, dim), beta.reshape(1, 1, dim))
